# baseline (speedup 1.0000x reference)
import functools

import jax
import jax.numpy as jnp
from jax import lax
from jax.experimental import pallas as pl
from jax.experimental.pallas import tpu as pltpu
from jax.experimental.pallas import tpu_sc as plsc

TOPK = 8
N_EXPERTS = 64
D_MODEL = 2048
N_ROWS = 16384
BM = 512
NC, NS, L = 2, 16, 16
NW = NC * NS
ROWS_PER_W = N_ROWS // NW
GROUPS_PER_W = ROWS_PER_W // L
N_CHUNKS = 1
CHUNK = N_ROWS // N_CHUNKS


def _logits_block(x_ref, w_ref, out_ref):
    w = jax.nn.sigmoid(w_ref[...])
    out_ref[...] = jax.lax.dot_general(
        w, x_ref[...],
        dimension_numbers=(((1,), (1,)), ((), ())),
        preferred_element_type=jnp.float32,
    )


def _tc_logits_t(x, weight_raw, chunk):
    blk_off = chunk * (CHUNK // BM)
    return pl.pallas_call(
        _logits_block,
        grid=(CHUNK // BM,),
        in_specs=[
            pl.BlockSpec((BM, D_MODEL), lambda i: (i + blk_off, 0)),
            pl.BlockSpec((N_EXPERTS, D_MODEL), lambda i: (0, 0)),
        ],
        out_specs=pl.BlockSpec((N_EXPERTS, BM), lambda i: (0, i)),
        out_shape=jax.ShapeDtypeStruct((N_EXPERTS, CHUNK), jnp.float32),
        compiler_params=pltpu.CompilerParams(
            dimension_semantics=("parallel",),
        ),
    )(x, weight_raw)


def _sc_topk_body(lt_hbm, probs_hbm, idx_hbm, blk_v, pout_v, iout_v, sem):
    wid = lax.axis_index("s") * NC + lax.axis_index("c")
    rows_per_w = CHUNK // NW
    base = wid * rows_per_w
    pltpu.sync_copy(lt_hbm.at[:, pl.ds(base, rows_per_w)], blk_v)

    def ce(a, b):
        c = a[0] > b[0]
        hk = jnp.where(c, a[0], b[0])
        lk = jnp.where(c, b[0], a[0])
        hi = jnp.where(c, a[1], b[1])
        li = jnp.where(c, b[1], a[1])
        return (hk, hi), (lk, li)

    SORT8 = [(0, 1), (2, 3), (4, 5), (6, 7), (0, 2), (1, 3), (4, 6), (5, 7),
             (1, 2), (5, 6), (0, 4), (3, 7), (1, 5), (2, 6), (1, 4), (3, 6),
             (2, 4), (3, 5), (3, 4)]
    CLEAN8 = [(0, 4), (1, 5), (2, 6), (3, 7), (0, 2), (1, 3), (4, 6), (5, 7),
              (0, 1), (2, 3), (4, 5), (6, 7)]

    def sort8(el):
        for a, b in SORT8:
            el[a], el[b] = ce(el[a], el[b])
        return el

    def group(g, carry):
        g16 = g * L

        def block(b):
            el = []
            for t in range(8):
                e = 8 * b + t
                el.append((blk_v[e, pl.ds(g16, L)],
                           jnp.full((L,), e, dtype=jnp.int32)))
            return sort8(el)

        run = block(0)
        for b in range(1, 8):
            nxt = block(b)
            mrg = []
            for j in range(TOPK):
                rk, ri = run[j]
                bk, bi = nxt[TOPK - 1 - j]
                c = rk > bk
                mrg.append((jnp.where(c, rk, bk), jnp.where(c, ri, bi)))
            for a, b2 in CLEAN8:
                mrg[a], mrg[b2] = ce(mrg[a], mrg[b2])
            run = mrg
        s = [run[j][0] for j in range(TOPK)]
        si = [run[j][1] for j in range(TOPK)]
        for _ in range(2):
            for j in range(TOPK - 1):
                eqt = s[j] == s[j + 1]
                lo = jnp.minimum(si[j], si[j + 1])
                hi2 = jnp.maximum(si[j], si[j + 1])
                si[j] = jnp.where(eqt, lo, si[j])
                si[j + 1] = jnp.where(eqt, hi2, si[j + 1])
        es = [jnp.exp(s[j] - s[0]) for j in range(TOPK)]
        tot = es[0]
        for j in range(1, TOPK):
            tot = tot + es[j]
        for j in range(TOPK):
            pout_v[j, pl.ds(g16, L)] = es[j] / tot
            iout_v[j, pl.ds(g16, L)] = si[j]
        return carry

    lax.fori_loop(0, rows_per_w // L, group, 0)

    pltpu.sync_copy(pout_v, probs_hbm.at[:, pl.ds(base, rows_per_w)])
    pltpu.sync_copy(iout_v, idx_hbm.at[:, pl.ds(base, rows_per_w)])


def _sc_topk(logits_t):
    mesh = plsc.VectorSubcoreMesh(core_axis_name="c", subcore_axis_name="s")
    f = functools.partial(
        pl.kernel,
        mesh=mesh,
        out_type=[
            jax.ShapeDtypeStruct((TOPK, CHUNK), jnp.float32),
            jax.ShapeDtypeStruct((TOPK, CHUNK), jnp.int32),
        ],
        scratch_types=[
            pltpu.VMEM((N_EXPERTS, CHUNK // NW), jnp.float32),
            pltpu.VMEM((TOPK, CHUNK // NW), jnp.float32),
            pltpu.VMEM((TOPK, CHUNK // NW), jnp.int32),
            pltpu.SemaphoreType.DMA,
        ],
    )(_sc_topk_body)
    return f(logits_t)


@jax.jit
def kernel(x, weight_raw):
    parts = []
    for k in range(N_CHUNKS):
        lt = _tc_logits_t(x, weight_raw, k)
        parts.append(_sc_topk(lt))
    probs_t = jnp.concatenate([p for p, _ in parts], axis=1)
    idx_t = jnp.concatenate([i for _, i in parts], axis=1)
    return (probs_t.T, idx_t.T)


# speedup vs baseline: 1.1106x; 1.1106x over previous
import functools

import jax
import jax.numpy as jnp
from jax import lax
from jax.experimental import pallas as pl
from jax.experimental.pallas import tpu as pltpu
from jax.experimental.pallas import tpu_sc as plsc

TOPK = 8
N_EXPERTS = 64
D_MODEL = 2048
N_ROWS = 16384
BM = 1024
NC, NS, L = 2, 16, 16
NW = NC * NS
ROWS_PER_W = N_ROWS // NW
GROUPS_PER_W = ROWS_PER_W // L
N_CHUNKS = 1
CHUNK = N_ROWS // N_CHUNKS


def _logits_block(x_ref, w_ref, out_ref):
    w = jax.nn.sigmoid(w_ref[...])
    out_ref[...] = jax.lax.dot_general(
        w, x_ref[...],
        dimension_numbers=(((1,), (1,)), ((), ())),
        preferred_element_type=jnp.float32,
    )


def _tc_logits_t(x, weight_raw, chunk):
    blk_off = chunk * (CHUNK // BM)
    return pl.pallas_call(
        _logits_block,
        grid=(CHUNK // BM,),
        in_specs=[
            pl.BlockSpec((BM, D_MODEL), lambda i: (i + blk_off, 0)),
            pl.BlockSpec((N_EXPERTS, D_MODEL), lambda i: (0, 0)),
        ],
        out_specs=pl.BlockSpec((N_EXPERTS, BM), lambda i: (0, i)),
        out_shape=jax.ShapeDtypeStruct((N_EXPERTS, CHUNK), jnp.float32),
        compiler_params=pltpu.CompilerParams(
            dimension_semantics=("parallel",),
        ),
    )(x, weight_raw)


def _sc_topk_body(lt_hbm, probs_hbm, idx_hbm, blk_v, pout_v, iout_v, sem):
    wid = lax.axis_index("s") * NC + lax.axis_index("c")
    rows_per_w = CHUNK // NW
    base = wid * rows_per_w
    pltpu.sync_copy(lt_hbm.at[:, pl.ds(base, rows_per_w)], blk_v)

    def ce(a, b):
        c = a[0] > b[0]
        hk = jnp.where(c, a[0], b[0])
        lk = jnp.where(c, b[0], a[0])
        hi = jnp.where(c, a[1], b[1])
        li = jnp.where(c, b[1], a[1])
        return (hk, hi), (lk, li)

    SORT8 = [(0, 1), (2, 3), (4, 5), (6, 7), (0, 2), (1, 3), (4, 6), (5, 7),
             (1, 2), (5, 6), (0, 4), (3, 7), (1, 5), (2, 6), (1, 4), (3, 6),
             (2, 4), (3, 5), (3, 4)]
    CLEAN8 = [(0, 4), (1, 5), (2, 6), (3, 7), (0, 2), (1, 3), (4, 6), (5, 7),
              (0, 1), (2, 3), (4, 5), (6, 7)]

    def sort8(el):
        for a, b in SORT8:
            el[a], el[b] = ce(el[a], el[b])
        return el

    def group(g, carry):
        g16 = g * L

        def block(b):
            el = []
            for t in range(8):
                e = 8 * b + t
                el.append((blk_v[e, pl.ds(g16, L)],
                           jnp.full((L,), e, dtype=jnp.int32)))
            return sort8(el)

        run = block(0)
        for b in range(1, 8):
            nxt = block(b)
            mrg = []
            for j in range(TOPK):
                rk, ri = run[j]
                bk, bi = nxt[TOPK - 1 - j]
                c = rk > bk
                mrg.append((jnp.where(c, rk, bk), jnp.where(c, ri, bi)))
            for a, b2 in CLEAN8:
                mrg[a], mrg[b2] = ce(mrg[a], mrg[b2])
            run = mrg
        s = [run[j][0] for j in range(TOPK)]
        si = [run[j][1] for j in range(TOPK)]
        for _ in range(2):
            for j in range(TOPK - 1):
                eqt = s[j] == s[j + 1]
                lo = jnp.minimum(si[j], si[j + 1])
                hi2 = jnp.maximum(si[j], si[j + 1])
                si[j] = jnp.where(eqt, lo, si[j])
                si[j + 1] = jnp.where(eqt, hi2, si[j + 1])
        es = [jnp.exp(s[j] - s[0]) for j in range(TOPK)]
        tot = es[0]
        for j in range(1, TOPK):
            tot = tot + es[j]
        for j in range(TOPK):
            pout_v[j, pl.ds(g16, L)] = es[j] / tot
            iout_v[j, pl.ds(g16, L)] = si[j]
        return carry

    lax.fori_loop(0, rows_per_w // L, group, 0)

    pltpu.sync_copy(pout_v, probs_hbm.at[:, pl.ds(base, rows_per_w)])
    pltpu.sync_copy(iout_v, idx_hbm.at[:, pl.ds(base, rows_per_w)])


def _sc_topk(logits_t):
    mesh = plsc.VectorSubcoreMesh(core_axis_name="c", subcore_axis_name="s")
    f = functools.partial(
        pl.kernel,
        mesh=mesh,
        out_type=[
            jax.ShapeDtypeStruct((TOPK, CHUNK), jnp.float32),
            jax.ShapeDtypeStruct((TOPK, CHUNK), jnp.int32),
        ],
        scratch_types=[
            pltpu.VMEM((N_EXPERTS, CHUNK // NW), jnp.float32),
            pltpu.VMEM((TOPK, CHUNK // NW), jnp.float32),
            pltpu.VMEM((TOPK, CHUNK // NW), jnp.int32),
            pltpu.SemaphoreType.DMA,
        ],
    )(_sc_topk_body)
    return f(logits_t)


@jax.jit
def kernel(x, weight_raw):
    parts = []
    for k in range(N_CHUNKS):
        lt = _tc_logits_t(x, weight_raw, k)
        parts.append(_sc_topk(lt))
    probs_t = jnp.concatenate([p for p, _ in parts], axis=1)
    idx_t = jnp.concatenate([i for _, i in parts], axis=1)
    return (probs_t.T, idx_t.T)
